# baseline (speedup 1.0000x reference)
import functools

import jax
import jax.numpy as jnp
from jax import lax
from jax.experimental import pallas as pl
from jax.experimental.pallas import tpu as pltpu
from jax.experimental.pallas import tpu_sc as plsc

N_NODES = 10000
N_PAD = 10240
NROW = N_PAD // 128
E = 320000
NC, NS, L = 2, 16, 16
NW = NC * NS
EPT = E // NW
KB = 80
NB = EPT // KB
NCH = 5
CB = NB // NCH
NSLOT = 3

_mesh = plsc.VectorSubcoreMesh(
    core_axis_name="c", subcore_axis_name="s", num_cores=NC, num_subcores=NS
)
_sc_params = pltpu.CompilerParams(needs_layout_passes=False)


@functools.partial(
    pl.kernel,
    out_type=[
        jax.ShapeDtypeStruct((NW, N_PAD), jnp.float32),
        jax.ShapeDtypeStruct((NW, N_PAD), jnp.float32),
    ],
    mesh=_mesh,
    scratch_types=[
        pltpu.VMEM((NB, KB), jnp.int32),
        pltpu.VMEM((NB, KB), jnp.int32),
        pltpu.VMEM((N_PAD,), jnp.float32),
        pltpu.VMEM((N_PAD,), jnp.float32),
    ],
    compiler_params=_sc_params,
)
def _deg_kernel(src_hbm, dst_hbm, sdeg_out, ddeg_out, src_v, dst_v, hs, hd):
    cid = lax.axis_index("c")
    sid = lax.axis_index("s")
    wid = cid * NS + sid

    def stage(c, _):
        pltpu.sync_copy(src_hbm.at[wid, c], src_v.at[pl.ds(c * CB, CB)])
        pltpu.sync_copy(dst_hbm.at[wid, c], dst_v.at[pl.ds(c * CB, CB)])
        return _

    lax.fori_loop(0, NCH, stage, 0)

    zero16 = jnp.zeros((L,), jnp.float32)

    def zpriv(i, _):
        hs[pl.ds(i * L, L)] = zero16
        hd[pl.ds(i * L, L)] = zero16
        return _

    lax.fori_loop(0, N_PAD // L, zpriv, 0)

    ones = jnp.ones((L,), jnp.float32)
    _PB = KB // L

    def hbody(i, _):
        bq = i // _PB
        o = (i % _PB) * L
        sv = src_v[bq, pl.ds(o, L)]
        dv = dst_v[bq, pl.ds(o, L)]
        plsc.addupdate_scatter(hs, [sv], ones)
        plsc.addupdate_scatter(hd, [dv], ones)
        return _

    lax.fori_loop(0, EPT // L, hbody, 0)

    pltpu.sync_copy(hs, sdeg_out.at[wid])
    pltpu.sync_copy(hd, ddeg_out.at[wid])


@functools.partial(
    pl.kernel,
    out_type=jax.ShapeDtypeStruct((NC, N_PAD, 128), jnp.float32),
    mesh=_mesh,
    scratch_types=[
        pltpu.VMEM((CB, KB), jnp.int32),
        pltpu.VMEM((CB, KB), jnp.int32),
        pltpu.VMEM((NSLOT * KB, 128), jnp.float32),
        pltpu.SemaphoreType.DMA,
        pltpu.SemaphoreType.DMA,
        pltpu.VMEM_SHARED((N_PAD, 128), jnp.float32),
    ],
    compiler_params=_sc_params,
)
def _agg_kernel(xs_hbm, src_hbm, dst_hbm, pooled_out,
                src_v, dst_v, buf, gsem, ssem, acc):
    cid = lax.axis_index("c")
    sid = lax.axis_index("s")
    wid = cid * NS + sid

    zero16 = jnp.zeros((L,), jnp.float32)

    def zbuf(i, _):
        r = i // 8
        c = (i % 8) * L
        buf[r, pl.ds(c, L)] = zero16
        return _

    lax.fori_loop(0, KB * 8, zbuf, 0)

    def zacc(i, _):
        pltpu.sync_copy(buf.at[pl.ds(0, KB)],
                        acc.at[pl.ds(sid * 640 + i * KB, KB)])
        return _

    lax.fori_loop(0, 640 // KB, zacc, 0)
    plsc.subcore_barrier()

    def gather(q, slot):
        return pltpu.async_copy(
            xs_hbm.at[src_v.at[q]], buf.at[pl.ds(slot * KB, KB)], gsem)

    def wait_scatter(q):
        pltpu.make_async_copy(
            buf.at[pl.ds((q % NSLOT) * KB, KB)],
            acc.at[dst_v.at[q]], ssem).wait()

    def chunk(ch, _):
        pltpu.sync_copy(src_hbm.at[wid, ch], src_v)
        pltpu.sync_copy(dst_hbm.at[wid, ch], dst_v)
        for s in range(NSLOT - 1):
            gather(s, s)

        def body(q, carry):
            slot = q % NSLOT
            @pl.when(q >= 1)
            def drain():
                wait_scatter(q - 1)

            @pl.when(q + NSLOT - 1 < CB)
            def refire():
                gather(q + NSLOT - 1, (q + NSLOT - 1) % NSLOT)

            pltpu.make_async_copy(
                xs_hbm.at[src_v.at[q]], buf.at[pl.ds(slot * KB, KB)], gsem
            ).wait()
            pltpu.async_copy(buf.at[pl.ds(slot * KB, KB)],
                             acc.at[dst_v.at[q]], ssem, add=True)
            return carry

        lax.fori_loop(0, CB, body, 0)
        wait_scatter(CB - 1)
        return _

    lax.fori_loop(0, NCH, chunk, 0)

    plsc.subcore_barrier()
    pltpu.sync_copy(acc.at[pl.ds(sid * 640, 640)],
                    pooled_out.at[cid, pl.ds(sid * 640, 640)])


def _scales_body(sdeg_ref, ddeg_ref, s_ref, r_ref):
    s_ref[...] = lax.rsqrt(jnp.maximum(jnp.sum(sdeg_ref[...], axis=0), 1.0))
    r_ref[...] = lax.rsqrt(jnp.maximum(jnp.sum(ddeg_ref[...], axis=0), 1.0))


_scales = pl.pallas_call(
    _scales_body,
    out_shape=[
        jax.ShapeDtypeStruct((NROW, 128), jnp.float32),
        jax.ShapeDtypeStruct((NROW, 128), jnp.float32),
    ],
)


_BS2 = 1024


def _scale_body(s_ref, x_ref, xs_ref):
    xs_ref[...] = x_ref[...] * s_ref[...]


_scale = pl.pallas_call(
    _scale_body,
    grid=(N_PAD // _BS2,),
    in_specs=[
        pl.BlockSpec((_BS2, 1), lambda i: (i, 0)),
        pl.BlockSpec((_BS2, 128), lambda i: (i, 0)),
    ],
    out_specs=pl.BlockSpec((_BS2, 128), lambda i: (i, 0)),
    out_shape=jax.ShapeDtypeStruct((N_PAD, 128), jnp.float32),
)


_BS4 = 1024


def _final_body(p_ref, r_ref, w_ref, b_ref, o_ref):
    pooled = (p_ref[0] + p_ref[1]) * r_ref[...]
    acc = jnp.dot(pooled, w_ref[...], preferred_element_type=jnp.float32)
    o_ref[...] = jnp.maximum(acc + b_ref[...], 0.0)


_final = pl.pallas_call(
    _final_body,
    grid=(N_PAD // _BS4,),
    in_specs=[
        pl.BlockSpec((NC, _BS4, 128), lambda i: (0, i, 0)),
        pl.BlockSpec((_BS4, 1), lambda i: (i, 0)),
        pl.BlockSpec((128, 128), lambda i: (0, 0)),
        pl.BlockSpec((1, 128), lambda i: (0, 0)),
    ],
    out_specs=pl.BlockSpec((_BS4, 128), lambda i: (i, 0)),
    out_shape=jax.ShapeDtypeStruct((N_PAD, 128), jnp.float32),
)


def kernel(x, edge_index, W, b):
    src_c = edge_index[0].astype(jnp.int32).reshape(NW, NCH, CB, KB)
    dst_c = edge_index[1].astype(jnp.int32).reshape(NW, NCH, CB, KB)
    x_pad = jnp.pad(x, ((0, N_PAD - N_NODES), (0, 0)))

    sdeg, ddeg = _deg_kernel(src_c, dst_c)
    s3, r3 = _scales(sdeg.reshape(NW, NROW, 128), ddeg.reshape(NW, NROW, 128))
    s_col = s3.reshape(N_PAD, 1)
    r_col = r3.reshape(N_PAD, 1)

    x_scaled = _scale(s_col, x_pad)
    pooled = _agg_kernel(x_scaled, src_c, dst_c)
    out = _final(pooled, r_col, W, b.reshape(1, 128))
    return out[:N_NODES]


# speedup vs baseline: 1.0584x; 1.0584x over previous
import functools

import jax
import jax.numpy as jnp
from jax import lax
from jax.experimental import pallas as pl
from jax.experimental.pallas import tpu as pltpu
from jax.experimental.pallas import tpu_sc as plsc

N_NODES = 10000
N_PAD = 10240
NROW = N_PAD // 128
E = 320000
NC, NS, L = 2, 16, 16
NW = NC * NS
EPT = E // NW
KB = 80
NB = EPT // KB
NCH = 5
CB = NB // NCH
NSLOT = 3

_mesh = plsc.VectorSubcoreMesh(
    core_axis_name="c", subcore_axis_name="s", num_cores=NC, num_subcores=NS
)
_sc_params = pltpu.CompilerParams(needs_layout_passes=False)


@functools.partial(
    pl.kernel,
    out_type=[
        jax.ShapeDtypeStruct((NW, N_PAD), jnp.float32),
        jax.ShapeDtypeStruct((NW, N_PAD), jnp.float32),
    ],
    mesh=_mesh,
    scratch_types=[
        pltpu.VMEM((NB, KB), jnp.int32),
        pltpu.VMEM((NB, KB), jnp.int32),
        pltpu.VMEM((N_PAD,), jnp.float32),
        pltpu.VMEM((N_PAD,), jnp.float32),
    ],
    compiler_params=_sc_params,
)
def _deg_kernel(edge_hbm, sdeg_out, ddeg_out, src_v, dst_v, hs, hd):
    cid = lax.axis_index("c")
    sid = lax.axis_index("s")
    wid = cid * NS + sid

    def stage(c, _):
        pltpu.sync_copy(edge_hbm.at[0, wid, c], src_v.at[pl.ds(c * CB, CB)])
        pltpu.sync_copy(edge_hbm.at[1, wid, c], dst_v.at[pl.ds(c * CB, CB)])
        return _

    lax.fori_loop(0, NCH, stage, 0)

    zero16 = jnp.zeros((L,), jnp.float32)

    def zpriv(i, _):
        hs[pl.ds(i * L, L)] = zero16
        hd[pl.ds(i * L, L)] = zero16
        return _

    lax.fori_loop(0, N_PAD // L, zpriv, 0)

    ones = jnp.ones((L,), jnp.float32)
    _PB = KB // L

    def hbody(i, _):
        bq = i // _PB
        o = (i % _PB) * L
        sv = src_v[bq, pl.ds(o, L)]
        dv = dst_v[bq, pl.ds(o, L)]
        plsc.addupdate_scatter(hs, [sv], ones)
        plsc.addupdate_scatter(hd, [dv], ones)
        return _

    lax.fori_loop(0, EPT // L, hbody, 0)

    pltpu.sync_copy(hs, sdeg_out.at[wid])
    pltpu.sync_copy(hd, ddeg_out.at[wid])


@functools.partial(
    pl.kernel,
    out_type=jax.ShapeDtypeStruct((NC, N_PAD, 128), jnp.float32),
    mesh=_mesh,
    scratch_types=[
        pltpu.VMEM((CB, KB), jnp.int32),
        pltpu.VMEM((CB, KB), jnp.int32),
        pltpu.VMEM((NSLOT * KB, 128), jnp.float32),
        pltpu.SemaphoreType.DMA,
        pltpu.SemaphoreType.DMA,
        pltpu.VMEM_SHARED((N_PAD, 128), jnp.float32),
    ],
    compiler_params=_sc_params,
)
def _agg_kernel(xs_hbm, edge_hbm, pooled_out,
                src_v, dst_v, buf, gsem, ssem, acc):
    cid = lax.axis_index("c")
    sid = lax.axis_index("s")
    wid = cid * NS + sid

    zero16 = jnp.zeros((L,), jnp.float32)

    def zbuf(i, _):
        r = i // 8
        c = (i % 8) * L
        buf[r, pl.ds(c, L)] = zero16
        return _

    lax.fori_loop(0, KB * 8, zbuf, 0)

    def zacc(i, _):
        pltpu.sync_copy(buf.at[pl.ds(0, KB)],
                        acc.at[pl.ds(sid * 640 + i * KB, KB)])
        return _

    lax.fori_loop(0, 640 // KB, zacc, 0)
    plsc.subcore_barrier()

    def gather(q, slot):
        return pltpu.async_copy(
            xs_hbm.at[src_v.at[q]], buf.at[pl.ds(slot * KB, KB)], gsem)

    def wait_scatter(q):
        pltpu.make_async_copy(
            buf.at[pl.ds((q % NSLOT) * KB, KB)],
            acc.at[dst_v.at[q]], ssem).wait()

    def chunk(ch, _):
        pltpu.sync_copy(edge_hbm.at[0, wid, ch], src_v)
        pltpu.sync_copy(edge_hbm.at[1, wid, ch], dst_v)
        for s in range(NSLOT - 1):
            gather(s, s)

        def body(q, carry):
            slot = q % NSLOT
            @pl.when(q >= 1)
            def drain():
                wait_scatter(q - 1)

            @pl.when(q + NSLOT - 1 < CB)
            def refire():
                gather(q + NSLOT - 1, (q + NSLOT - 1) % NSLOT)

            pltpu.make_async_copy(
                xs_hbm.at[src_v.at[q]], buf.at[pl.ds(slot * KB, KB)], gsem
            ).wait()
            pltpu.async_copy(buf.at[pl.ds(slot * KB, KB)],
                             acc.at[dst_v.at[q]], ssem, add=True)
            return carry

        lax.fori_loop(0, CB, body, 0)
        wait_scatter(CB - 1)
        return _

    lax.fori_loop(0, NCH, chunk, 0)

    plsc.subcore_barrier()
    pltpu.sync_copy(acc.at[pl.ds(sid * 640, 640)],
                    pooled_out.at[cid, pl.ds(sid * 640, 640)])


def _scales_body(sdeg_ref, ddeg_ref, s_ref, r_ref):
    s_ref[...] = lax.rsqrt(jnp.maximum(jnp.sum(sdeg_ref[...], axis=0), 1.0))
    r_ref[...] = lax.rsqrt(jnp.maximum(jnp.sum(ddeg_ref[...], axis=0), 1.0))


_scales = pl.pallas_call(
    _scales_body,
    out_shape=[
        jax.ShapeDtypeStruct((NROW, 128), jnp.float32),
        jax.ShapeDtypeStruct((NROW, 128), jnp.float32),
    ],
)


_BS2 = 1024


def _scale_body(s_ref, x_ref, xs_ref):
    xs_ref[...] = x_ref[...] * s_ref[...]


_scale = pl.pallas_call(
    _scale_body,
    grid=(N_PAD // _BS2,),
    in_specs=[
        pl.BlockSpec((_BS2, 1), lambda i: (i, 0)),
        pl.BlockSpec((_BS2, 128), lambda i: (i, 0)),
    ],
    out_specs=pl.BlockSpec((_BS2, 128), lambda i: (i, 0)),
    out_shape=jax.ShapeDtypeStruct((N_PAD, 128), jnp.float32),
)


_BS4 = 1024


def _final_body(p_ref, r_ref, w_ref, b_ref, o_ref):
    pooled = (p_ref[0] + p_ref[1]) * r_ref[...]
    acc = jnp.dot(pooled, w_ref[...], preferred_element_type=jnp.float32)
    o_ref[...] = jnp.maximum(acc + b_ref[...], 0.0)


_final = pl.pallas_call(
    _final_body,
    grid=(N_PAD // _BS4,),
    in_specs=[
        pl.BlockSpec((NC, _BS4, 128), lambda i: (0, i, 0)),
        pl.BlockSpec((_BS4, 1), lambda i: (i, 0)),
        pl.BlockSpec((128, 128), lambda i: (0, 0)),
        pl.BlockSpec((1, 128), lambda i: (0, 0)),
    ],
    out_specs=pl.BlockSpec((_BS4, 128), lambda i: (i, 0)),
    out_shape=jax.ShapeDtypeStruct((N_PAD, 128), jnp.float32),
)


def kernel(x, edge_index, W, b):
    edge_c = edge_index.astype(jnp.int32).reshape(2, NW, NCH, CB, KB)
    x_pad = jnp.pad(x, ((0, N_PAD - N_NODES), (0, 0)))

    sdeg, ddeg = _deg_kernel(edge_c)
    s3, r3 = _scales(sdeg.reshape(NW, NROW, 128), ddeg.reshape(NW, NROW, 128))
    s_col = s3.reshape(N_PAD, 1)
    r_col = r3.reshape(N_PAD, 1)

    x_scaled = _scale(s_col, x_pad)
    pooled = _agg_kernel(x_scaled, edge_c)
    out = _final(pooled, r_col, W, b.reshape(1, 128))
    return out[:N_NODES]


# speedup vs baseline: 1.0879x; 1.0278x over previous
import functools

import jax
import jax.numpy as jnp
from jax import lax
from jax.experimental import pallas as pl
from jax.experimental.pallas import tpu as pltpu
from jax.experimental.pallas import tpu_sc as plsc

N_NODES = 10000
N_PAD = 10240
NROW = N_PAD // 128
E = 320000
NC, NS, L = 2, 16, 16
NW = NC * NS
EPT = E // NW
KB = 80
NB = EPT // KB
NCH = 5
CB = NB // NCH
NSLOT = 3

_mesh = plsc.VectorSubcoreMesh(
    core_axis_name="c", subcore_axis_name="s", num_cores=NC, num_subcores=NS
)
_sc_params = pltpu.CompilerParams(needs_layout_passes=False)


@functools.partial(
    pl.kernel,
    out_type=[
        jax.ShapeDtypeStruct((NW, N_PAD), jnp.float32),
        jax.ShapeDtypeStruct((NW, N_PAD), jnp.float32),
    ],
    mesh=_mesh,
    scratch_types=[
        pltpu.VMEM((NB, KB), jnp.int32),
        pltpu.VMEM((NB, KB), jnp.int32),
        pltpu.VMEM((N_PAD,), jnp.float32),
        pltpu.VMEM((N_PAD,), jnp.float32),
    ],
    compiler_params=_sc_params,
)
def _deg_kernel(edge_hbm, sdeg_out, ddeg_out, src_v, dst_v, hs, hd):
    cid = lax.axis_index("c")
    sid = lax.axis_index("s")
    wid = cid * NS + sid

    def stage(c, _):
        pltpu.sync_copy(edge_hbm.at[0, wid, c], src_v.at[pl.ds(c * CB, CB)])
        pltpu.sync_copy(edge_hbm.at[1, wid, c], dst_v.at[pl.ds(c * CB, CB)])
        return _

    lax.fori_loop(0, NCH, stage, 0)

    zero16 = jnp.zeros((L,), jnp.float32)

    def zpriv(i, _):
        hs[pl.ds(i * L, L)] = zero16
        hd[pl.ds(i * L, L)] = zero16
        return _

    lax.fori_loop(0, N_PAD // L, zpriv, 0)

    ones = jnp.ones((L,), jnp.float32)
    _PB = KB // L

    def hbody(i, _):
        bq = i // _PB
        o = (i % _PB) * L
        sv = src_v[bq, pl.ds(o, L)]
        dv = dst_v[bq, pl.ds(o, L)]
        plsc.addupdate_scatter(hs, [sv], ones)
        plsc.addupdate_scatter(hd, [dv], ones)
        return _

    lax.fori_loop(0, EPT // L, hbody, 0)

    pltpu.sync_copy(hs, sdeg_out.at[wid])
    pltpu.sync_copy(hd, ddeg_out.at[wid])


@functools.partial(
    pl.kernel,
    out_type=jax.ShapeDtypeStruct((NC, N_PAD, 128), jnp.float32),
    mesh=_mesh,
    scratch_types=[
        pltpu.VMEM((CB, KB), jnp.int32),
        pltpu.VMEM((CB, KB), jnp.int32),
        pltpu.VMEM((NSLOT * KB, 128), jnp.float32),
        pltpu.SemaphoreType.DMA,
        pltpu.SemaphoreType.DMA,
        pltpu.VMEM_SHARED((N_PAD, 128), jnp.float32),
    ],
    compiler_params=_sc_params,
)
def _agg_kernel(xs_hbm, edge_hbm, pooled_out,
                src_v, dst_v, buf, gsem, ssem, acc):
    cid = lax.axis_index("c")
    sid = lax.axis_index("s")
    wid = cid * NS + sid

    zero16 = jnp.zeros((L,), jnp.float32)

    def zbuf(i, _):
        r = i // 8
        c = (i % 8) * L
        buf[r, pl.ds(c, L)] = zero16
        return _

    lax.fori_loop(0, KB * 8, zbuf, 0)

    def zacc(i, _):
        pltpu.sync_copy(buf.at[pl.ds(0, KB)],
                        acc.at[pl.ds(sid * 640 + i * KB, KB)])
        return _

    lax.fori_loop(0, 640 // KB, zacc, 0)
    plsc.subcore_barrier()

    def gather(q, slot):
        return pltpu.async_copy(
            xs_hbm.at[src_v.at[q]], buf.at[pl.ds(slot * KB, KB)], gsem)

    def wait_scatter(q):
        pltpu.make_async_copy(
            buf.at[pl.ds((q % NSLOT) * KB, KB)],
            acc.at[dst_v.at[q]], ssem).wait()

    def chunk(ch, _):
        pltpu.sync_copy(edge_hbm.at[0, wid, ch], src_v)
        pltpu.sync_copy(edge_hbm.at[1, wid, ch], dst_v)
        for s in range(NSLOT - 1):
            gather(s, s)

        def body(q, carry):
            slot = q % NSLOT
            @pl.when(q >= 1)
            def drain():
                wait_scatter(q - 1)

            @pl.when(q + NSLOT - 1 < CB)
            def refire():
                gather(q + NSLOT - 1, (q + NSLOT - 1) % NSLOT)

            pltpu.make_async_copy(
                xs_hbm.at[src_v.at[q]], buf.at[pl.ds(slot * KB, KB)], gsem
            ).wait()
            pltpu.async_copy(buf.at[pl.ds(slot * KB, KB)],
                             acc.at[dst_v.at[q]], ssem, add=True)
            return carry

        lax.fori_loop(0, CB, body, 0)
        wait_scatter(CB - 1)
        return _

    lax.fori_loop(0, NCH, chunk, 0)

    plsc.subcore_barrier()
    pltpu.sync_copy(acc.at[pl.ds(sid * 640, 640)],
                    pooled_out.at[cid, pl.ds(sid * 640, 640)])


def _scales_body(sdeg_ref, ddeg_ref, s_ref, r_ref):
    s_ref[...] = lax.rsqrt(jnp.maximum(jnp.sum(sdeg_ref[...], axis=0), 1.0))
    r_ref[...] = lax.rsqrt(jnp.maximum(jnp.sum(ddeg_ref[...], axis=0), 1.0))


_scales = pl.pallas_call(
    _scales_body,
    out_shape=[
        jax.ShapeDtypeStruct((NROW, 128), jnp.float32),
        jax.ShapeDtypeStruct((NROW, 128), jnp.float32),
    ],
)


_BS2 = 2048


def _scale_body(s_ref, x_ref, xs_ref):
    xs_ref[...] = x_ref[...] * s_ref[...]


_scale = pl.pallas_call(
    _scale_body,
    grid=(N_PAD // _BS2,),
    in_specs=[
        pl.BlockSpec((_BS2, 1), lambda i: (i, 0)),
        pl.BlockSpec((_BS2, 128), lambda i: (i, 0)),
    ],
    out_specs=pl.BlockSpec((_BS2, 128), lambda i: (i, 0)),
    out_shape=jax.ShapeDtypeStruct((N_PAD, 128), jnp.float32),
)


_BS4 = 2048


def _final_body(p_ref, r_ref, w_ref, b_ref, o_ref):
    pooled = (p_ref[0] + p_ref[1]) * r_ref[...]
    acc = jnp.dot(pooled, w_ref[...], preferred_element_type=jnp.float32)
    o_ref[...] = jnp.maximum(acc + b_ref[...], 0.0)


_final = pl.pallas_call(
    _final_body,
    grid=(N_PAD // _BS4,),
    in_specs=[
        pl.BlockSpec((NC, _BS4, 128), lambda i: (0, i, 0)),
        pl.BlockSpec((_BS4, 1), lambda i: (i, 0)),
        pl.BlockSpec((128, 128), lambda i: (0, 0)),
        pl.BlockSpec((1, 128), lambda i: (0, 0)),
    ],
    out_specs=pl.BlockSpec((_BS4, 128), lambda i: (i, 0)),
    out_shape=jax.ShapeDtypeStruct((N_PAD, 128), jnp.float32),
)


def kernel(x, edge_index, W, b):
    edge_c = edge_index.astype(jnp.int32).reshape(2, NW, NCH, CB, KB)
    x_pad = jnp.pad(x, ((0, N_PAD - N_NODES), (0, 0)))

    sdeg, ddeg = _deg_kernel(edge_c)
    s3, r3 = _scales(sdeg.reshape(NW, NROW, 128), ddeg.reshape(NW, NROW, 128))
    s_col = s3.reshape(N_PAD, 1)
    r_col = r3.reshape(N_PAD, 1)

    x_scaled = _scale(s_col, x_pad)
    pooled = _agg_kernel(x_scaled, edge_c)
    out = _final(pooled, r_col, W, b.reshape(1, 128))
    return out[:N_NODES]


# speedup vs baseline: 1.1323x; 1.0408x over previous
import functools

import jax
import jax.numpy as jnp
from jax import lax
from jax.experimental import pallas as pl
from jax.experimental.pallas import tpu as pltpu
from jax.experimental.pallas import tpu_sc as plsc

N_NODES = 10000
N_PAD = 10240
NROW = N_PAD // 128
E = 320000
NC, NS, L = 2, 16, 16
NW = NC * NS
EPT = E // NW
KB = 80
NB = EPT // KB
NCH = 5
CB = NB // NCH
NSLOT = 3

_mesh = plsc.VectorSubcoreMesh(
    core_axis_name="c", subcore_axis_name="s", num_cores=NC, num_subcores=NS
)
_sc_params = pltpu.CompilerParams(needs_layout_passes=False)


@functools.partial(
    pl.kernel,
    out_type=[
        jax.ShapeDtypeStruct((NW, N_PAD), jnp.float32),
        jax.ShapeDtypeStruct((NW, N_PAD), jnp.float32),
    ],
    mesh=_mesh,
    scratch_types=[
        pltpu.VMEM((NB, KB), jnp.int32),
        pltpu.VMEM((NB, KB), jnp.int32),
        pltpu.VMEM((N_PAD,), jnp.float32),
        pltpu.VMEM((N_PAD,), jnp.float32),
        pltpu.SemaphoreType.DMA,
    ],
    compiler_params=_sc_params,
)
def _deg_kernel(edge_hbm, sdeg_out, ddeg_out, src_v, dst_v, hs, hd, tsem):
    cid = lax.axis_index("c")
    sid = lax.axis_index("s")
    wid = cid * NS + sid

    def stage(c, _):
        pltpu.async_copy(edge_hbm.at[0, wid, c], src_v.at[pl.ds(c * CB, CB)], tsem)
        pltpu.async_copy(edge_hbm.at[1, wid, c], dst_v.at[pl.ds(c * CB, CB)], tsem)
        return _

    lax.fori_loop(0, NCH, stage, 0)

    zero16 = jnp.zeros((L,), jnp.float32)

    def zpriv(i, _):
        hs[pl.ds(i * L, L)] = zero16
        hd[pl.ds(i * L, L)] = zero16
        return _

    lax.fori_loop(0, N_PAD // L, zpriv, 0)

    def drain(c, _):
        pltpu.make_async_copy(
            edge_hbm.at[0, wid, c], src_v.at[pl.ds(c * CB, CB)], tsem).wait()
        pltpu.make_async_copy(
            edge_hbm.at[1, wid, c], dst_v.at[pl.ds(c * CB, CB)], tsem).wait()
        return _

    lax.fori_loop(0, NCH, drain, 0)

    ones = jnp.ones((L,), jnp.float32)
    _PB = KB // L

    def hbody(i, _):
        bq = i // _PB
        o = (i % _PB) * L
        sv = src_v[bq, pl.ds(o, L)]
        dv = dst_v[bq, pl.ds(o, L)]
        plsc.addupdate_scatter(hs, [sv], ones)
        plsc.addupdate_scatter(hd, [dv], ones)
        return _

    lax.fori_loop(0, EPT // L, hbody, 0)

    pltpu.sync_copy(hs, sdeg_out.at[wid])
    pltpu.sync_copy(hd, ddeg_out.at[wid])


@functools.partial(
    pl.kernel,
    out_type=jax.ShapeDtypeStruct((NC, N_PAD, 128), jnp.float32),
    mesh=_mesh,
    scratch_types=[
        pltpu.VMEM((CB, KB), jnp.int32),
        pltpu.VMEM((CB, KB), jnp.int32),
        pltpu.VMEM((NSLOT * KB, 128), jnp.float32),
        pltpu.SemaphoreType.DMA,
        pltpu.SemaphoreType.DMA,
        pltpu.VMEM_SHARED((N_PAD, 128), jnp.float32),
    ],
    compiler_params=_sc_params,
)
def _agg_kernel(xs_hbm, edge_hbm, pooled_out,
                src_v, dst_v, buf, gsem, ssem, acc):
    cid = lax.axis_index("c")
    sid = lax.axis_index("s")
    wid = cid * NS + sid

    zero16 = jnp.zeros((L,), jnp.float32)

    def zbuf(i, _):
        r = i // 8
        c = (i % 8) * L
        buf[r, pl.ds(c, L)] = zero16
        return _

    lax.fori_loop(0, KB * 8, zbuf, 0)

    def zacc(i, _):
        pltpu.sync_copy(buf.at[pl.ds(0, KB)],
                        acc.at[pl.ds(sid * 640 + i * KB, KB)])
        return _

    lax.fori_loop(0, 640 // KB, zacc, 0)
    plsc.subcore_barrier()

    def gather(q, slot):
        return pltpu.async_copy(
            xs_hbm.at[src_v.at[q]], buf.at[pl.ds(slot * KB, KB)], gsem)

    def wait_scatter(q):
        pltpu.make_async_copy(
            buf.at[pl.ds((q % NSLOT) * KB, KB)],
            acc.at[dst_v.at[q]], ssem).wait()

    def chunk(ch, _):
        pltpu.sync_copy(edge_hbm.at[0, wid, ch], src_v)
        pltpu.sync_copy(edge_hbm.at[1, wid, ch], dst_v)
        for s in range(NSLOT - 1):
            gather(s, s)

        def body(q, carry):
            slot = q % NSLOT
            @pl.when(q >= 1)
            def drain():
                wait_scatter(q - 1)

            @pl.when(q + NSLOT - 1 < CB)
            def refire():
                gather(q + NSLOT - 1, (q + NSLOT - 1) % NSLOT)

            pltpu.make_async_copy(
                xs_hbm.at[src_v.at[q]], buf.at[pl.ds(slot * KB, KB)], gsem
            ).wait()
            pltpu.async_copy(buf.at[pl.ds(slot * KB, KB)],
                             acc.at[dst_v.at[q]], ssem, add=True)
            return carry

        lax.fori_loop(0, CB, body, 0)
        wait_scatter(CB - 1)
        return _

    lax.fori_loop(0, NCH, chunk, 0)

    plsc.subcore_barrier()
    pltpu.sync_copy(acc.at[pl.ds(sid * 640, 640)],
                    pooled_out.at[cid, pl.ds(sid * 640, 640)])


def _scales_body(sdeg_ref, ddeg_ref, s_ref, r_ref):
    s_ref[...] = lax.rsqrt(jnp.maximum(jnp.sum(sdeg_ref[...], axis=0), 1.0))
    r_ref[...] = lax.rsqrt(jnp.maximum(jnp.sum(ddeg_ref[...], axis=0), 1.0))


_scales = pl.pallas_call(
    _scales_body,
    out_shape=[
        jax.ShapeDtypeStruct((NROW, 128), jnp.float32),
        jax.ShapeDtypeStruct((NROW, 128), jnp.float32),
    ],
)


_BS2 = 2048


def _scale_body(s_ref, x_ref, xs_ref):
    xs_ref[...] = x_ref[...] * s_ref[...]


_scale = pl.pallas_call(
    _scale_body,
    grid=(N_PAD // _BS2,),
    in_specs=[
        pl.BlockSpec((_BS2, 1), lambda i: (i, 0)),
        pl.BlockSpec((_BS2, 128), lambda i: (i, 0)),
    ],
    out_specs=pl.BlockSpec((_BS2, 128), lambda i: (i, 0)),
    out_shape=jax.ShapeDtypeStruct((N_PAD, 128), jnp.float32),
)


_BS4 = 2048


def _final_body(p_ref, r_ref, w_ref, b_ref, o_ref):
    pooled = (p_ref[0] + p_ref[1]) * r_ref[...]
    acc = jnp.dot(pooled, w_ref[...], preferred_element_type=jnp.float32)
    o_ref[...] = jnp.maximum(acc + b_ref[...], 0.0)


_final = pl.pallas_call(
    _final_body,
    grid=(N_PAD // _BS4,),
    in_specs=[
        pl.BlockSpec((NC, _BS4, 128), lambda i: (0, i, 0)),
        pl.BlockSpec((_BS4, 1), lambda i: (i, 0)),
        pl.BlockSpec((128, 128), lambda i: (0, 0)),
        pl.BlockSpec((1, 128), lambda i: (0, 0)),
    ],
    out_specs=pl.BlockSpec((_BS4, 128), lambda i: (i, 0)),
    out_shape=jax.ShapeDtypeStruct((N_PAD, 128), jnp.float32),
)


def kernel(x, edge_index, W, b):
    edge_c = edge_index.astype(jnp.int32).reshape(2, NW, NCH, CB, KB)
    x_pad = jnp.pad(x, ((0, N_PAD - N_NODES), (0, 0)))

    sdeg, ddeg = _deg_kernel(edge_c)
    s3, r3 = _scales(sdeg.reshape(NW, NROW, 128), ddeg.reshape(NW, NROW, 128))
    s_col = s3.reshape(N_PAD, 1)
    r_col = r3.reshape(N_PAD, 1)

    x_scaled = _scale(s_col, x_pad)
    pooled = _agg_kernel(x_scaled, edge_c)
    out = _final(pooled, r_col, W, b.reshape(1, 128))
    return out[:N_NODES]
